# baseline (speedup 1.0000x reference)
import functools

import jax
import jax.numpy as jnp
from jax import lax
from jax.experimental import pallas as pl
from jax.experimental.pallas import tpu as pltpu
from jax.experimental.pallas import tpu_sc as plsc

HIDDEN_DIM = 128
NUM_ROWS = 1001
BATCH = 16384

_NC = 2
_NS = 16
_NW = _NC * _NS
_BPW = BATCH // _NW
_CH = 256
_NCH = _BPW // _CH
_RPS = 64
_RTL = NUM_ROWS - 15 * _RPS

_mesh = plsc.VectorSubcoreMesh(core_axis_name="c", subcore_axis_name="s")


@functools.partial(
    pl.kernel,
    mesh=_mesh,
    out_type=jax.ShapeDtypeStruct((BATCH, HIDDEN_DIM), jnp.float32),
    scratch_types=[
        pltpu.VMEM((_BPW,), jnp.int32),
        pltpu.VMEM((_BPW, HIDDEN_DIM), jnp.float32),
        pltpu.VMEM_SHARED((NUM_ROWS, HIDDEN_DIM), jnp.float32),
        pltpu.SemaphoreType.DMA,
        pltpu.SemaphoreType.DMA,
    ],
)
def _embed(labels_hbm, table_hbm, out_hbm, idx_v, rows_v, table_sh, gsem, wsem):
    cid = lax.axis_index("c")
    sid = lax.axis_index("s")
    wid = sid * _NC + cid
    base = wid * _BPW
    pltpu.sync_copy(labels_hbm.at[pl.ds(base, _BPW)], idx_v)
    row0 = sid * _RPS

    @pl.when(sid < 15)
    def _stage():
        pltpu.sync_copy(table_hbm.at[pl.ds(row0, _RPS)],
                        table_sh.at[pl.ds(row0, _RPS)])

    @pl.when(sid == 15)
    def _stage_tail():
        pltpu.sync_copy(table_hbm.at[pl.ds(15 * _RPS, _RTL)],
                        table_sh.at[pl.ds(15 * _RPS, _RTL)])

    plsc.subcore_barrier()
    gathers = []
    for j in range(_NCH):
        gathers.append(
            pltpu.async_copy(
                table_sh.at[idx_v.at[pl.ds(j * _CH, _CH)]],
                rows_v.at[pl.ds(j * _CH, _CH)],
                gsem,
            )
        )
    writes = []
    for j in range(_NCH):
        gathers[j].wait()
        writes.append(
            pltpu.async_copy(
                rows_v.at[pl.ds(j * _CH, _CH)],
                out_hbm.at[pl.ds(base + j * _CH, _CH)],
                wsem,
            )
        )
    for w in writes:
        w.wait()


def kernel(labels, embedding):
    return _embed(labels.astype(jnp.int32), embedding)


# speedup vs baseline: 1.0325x; 1.0325x over previous
import functools

import jax
import jax.numpy as jnp
from jax import lax
from jax.experimental import pallas as pl
from jax.experimental.pallas import tpu as pltpu
from jax.experimental.pallas import tpu_sc as plsc

HIDDEN_DIM = 128
NUM_ROWS = 1001
BATCH = 16384

_NC = 2
_NS = 16
_NW = _NC * _NS
_BPW = BATCH // _NW
_CH = 128
_NCH = _BPW // _CH
_RPS = 64
_RTL = NUM_ROWS - 15 * _RPS

_mesh = plsc.VectorSubcoreMesh(core_axis_name="c", subcore_axis_name="s")


@functools.partial(
    pl.kernel,
    mesh=_mesh,
    out_type=jax.ShapeDtypeStruct((BATCH, HIDDEN_DIM), jnp.float32),
    scratch_types=[
        pltpu.VMEM((_BPW,), jnp.int32),
        pltpu.VMEM((_BPW, HIDDEN_DIM), jnp.float32),
        pltpu.VMEM_SHARED((NUM_ROWS, HIDDEN_DIM), jnp.float32),
        pltpu.SemaphoreType.DMA,
        pltpu.SemaphoreType.DMA,
        pltpu.SemaphoreType.DMA,
    ],
)
def _embed(labels_hbm, table_hbm, out_hbm, idx_v, rows_v, table_sh,
           lsem, gsem, wsem):
    cid = lax.axis_index("c")
    sid = lax.axis_index("s")
    wid = sid * _NC + cid
    base = wid * _BPW
    lcopy = pltpu.async_copy(labels_hbm.at[pl.ds(base, _BPW)], idx_v, lsem)
    row0 = sid * _RPS

    @pl.when(sid < 15)
    def _stage():
        pltpu.sync_copy(table_hbm.at[pl.ds(row0, _RPS)],
                        table_sh.at[pl.ds(row0, _RPS)])

    @pl.when(sid == 15)
    def _stage_tail():
        pltpu.sync_copy(table_hbm.at[pl.ds(15 * _RPS, _RTL)],
                        table_sh.at[pl.ds(15 * _RPS, _RTL)])

    lcopy.wait()
    plsc.subcore_barrier()
    gathers = []
    for j in range(_NCH):
        gathers.append(
            pltpu.async_copy(
                table_sh.at[idx_v.at[pl.ds(j * _CH, _CH)]],
                rows_v.at[pl.ds(j * _CH, _CH)],
                gsem,
            )
        )
    writes = []
    for j in range(_NCH):
        gathers[j].wait()
        writes.append(
            pltpu.async_copy(
                rows_v.at[pl.ds(j * _CH, _CH)],
                out_hbm.at[pl.ds(base + j * _CH, _CH)],
                wsem,
            )
        )
    for w in writes:
        w.wait()


def kernel(labels, embedding):
    return _embed(labels.astype(jnp.int32), embedding)
